# baseline (speedup 1.0000x reference)
import functools

import jax
import jax.numpy as jnp
from jax import lax
from jax.experimental import pallas as pl
from jax.experimental.pallas import tpu as pltpu
from jax.experimental.pallas import tpu_sc as plsc

B = 4096
F = 26
V = 100000
D = 64
BF = B * F

NC = 2
NS = 16
NW = NC * NS
BPW = BF // NW
CHUNK = 832
NCHUNK = BPW // CHUNK
LANES = 16


def _emb_kernel(x_hbm, tab_hbm, out_hbm, x_v, idx_v, rows_v, sem):
  wid = lax.axis_index("s") * NC + lax.axis_index("c")
  base = wid * BPW

  def chunk_body(c, _):
    gbase = base + c * CHUNK
    pltpu.sync_copy(x_hbm.at[pl.ds(gbase, CHUNK)], x_v)

    def idx_body(i, _):
      p16 = lax.iota(jnp.int32, LANES) + (gbase + i * LANES)
      f16 = lax.rem(p16, F)
      idx_v[pl.ds(i * LANES, LANES)] = x_v[pl.ds(i * LANES, LANES)] + f16 * V
      return 0

    lax.fori_loop(0, CHUNK // LANES, idx_body, 0, unroll=4)

    pltpu.async_copy(tab_hbm.at[idx_v], rows_v, sem).wait()
    pltpu.sync_copy(rows_v, out_hbm.at[pl.ds(gbase, CHUNK)])
    return 0

  lax.fori_loop(0, NCHUNK, chunk_body, 0)


@jax.jit
def _emb(x_flat, tab_flat):
  mesh = plsc.VectorSubcoreMesh(
      core_axis_name="c", subcore_axis_name="s", num_cores=NC, num_subcores=NS
  )
  return pl.kernel(
      _emb_kernel,
      out_type=jax.ShapeDtypeStruct((BF, D), jnp.float32),
      mesh=mesh,
      scratch_types=[
          pltpu.VMEM((CHUNK,), jnp.int32),
          pltpu.VMEM((CHUNK,), jnp.int32),
          pltpu.VMEM((CHUNK, D), jnp.float32),
          pltpu.SemaphoreType.DMA,
      ],
      compiler_params=pltpu.CompilerParams(use_tc_tiling_on_sc=False),
  )(x_flat, tab_flat)


def kernel(X, tables):
  x_flat = X.reshape(BF)
  tab_flat = tables.reshape(F * V, D)
  return _emb(x_flat, tab_flat).reshape(B, F, D)


# speedup vs baseline: 3.6389x; 3.6389x over previous
import functools

import jax
import jax.numpy as jnp
from jax import lax
from jax.experimental import pallas as pl
from jax.experimental.pallas import tpu as pltpu
from jax.experimental.pallas import tpu_sc as plsc

B = 4096
F = 26
V = 100000
D = 64

NC = 2
NS = 16
NW = NC * NS
LANES = 16

H0 = 50048
H1 = 49920
VT = 99968
NT = 2 * F


def _gather_row(idx_v, rowA, rowB, ob):
  def body(i, pos):
    v16 = idx_v[pl.ds(i * LANES, LANES)]
    m = v16 < H0
    g0 = plsc.load_gather(rowA, [v16], mask=m)
    g1 = plsc.load_gather(rowB, [v16 - H0], mask=~m)
    g = lax.select(m, g0, g1)
    plsc.store_scatter(ob, [pos], g)
    return pos + LANES

  lax.fori_loop(0, B // LANES, body, lax.iota(jnp.int32, LANES), unroll=4)


def _emb2_kernel(xt_hbm, tab_hbm, tail_hbm, out_hbm, idx_v, rowA, rowB, ob,
                 semA, semB):
  w = lax.axis_index("s") * NC + lax.axis_index("c")

  def fire(f, d):
    pltpu.async_copy(tab_hbm.at[f, d, pl.ds(0, H0)], rowA, semA)
    pltpu.async_copy(tab_hbm.at[f, d, pl.ds(H0, H1)], rowB.at[pl.ds(0, H1)],
                     semB)
    pltpu.async_copy(tail_hbm.at[f, d], rowB.at[pl.ds(H1, 128)], semB)

  fire(0, w)

  def task_body(t, _):
    f = t // 2
    d = (t % 2) * NW + w

    @pl.when(t % 2 == 0)
    def _():
      pltpu.sync_copy(xt_hbm.at[f], idx_v)

    pltpu.make_async_copy(tab_hbm.at[f, d, pl.ds(0, H0)], rowA, semA).wait()
    pltpu.make_async_copy(tab_hbm.at[f, d, pl.ds(H0, H1)],
                          rowB.at[pl.ds(0, H1)], semB).wait()
    pltpu.make_async_copy(tail_hbm.at[f, d], rowB.at[pl.ds(H1, 128)],
                          semB).wait()

    _gather_row(idx_v, rowA, rowB, ob)

    @pl.when(t + 1 < NT)
    def _():
      tn = t + 1
      fire(tn // 2, (tn % 2) * NW + w)

    pltpu.sync_copy(ob, out_hbm.at[f, d])
    return 0

  lax.fori_loop(0, NT, task_body, 0)


@jax.jit
def _emb2(xt, tab_p, tab_tail):
  mesh = plsc.VectorSubcoreMesh(
      core_axis_name="c", subcore_axis_name="s", num_cores=NC, num_subcores=NS
  )
  return pl.kernel(
      _emb2_kernel,
      out_type=jax.ShapeDtypeStruct((F, D, B), jnp.float32),
      mesh=mesh,
      scratch_types=[
          pltpu.VMEM((B,), jnp.int32),
          pltpu.VMEM((H0,), jnp.float32),
          pltpu.VMEM((H1 + 128,), jnp.float32),
          pltpu.VMEM((B,), jnp.float32),
          pltpu.SemaphoreType.DMA,
          pltpu.SemaphoreType.DMA,
      ],
      compiler_params=pltpu.CompilerParams(needs_layout_passes=False),
  )(xt, tab_p, tab_tail)


def kernel(X, tables):
  xt = X.T
  tab_p = tables.transpose(0, 2, 1)
  tab_tail = jnp.pad(tables[:, VT:, :].transpose(0, 2, 1), ((0, 0), (0, 0), (0, 96)))
  out_p = _emb2(xt, tab_p, tab_tail)
  return out_p.transpose(2, 0, 1)


# speedup vs baseline: 4.5124x; 1.2400x over previous
import functools

import jax
import jax.numpy as jnp
from jax import lax
from jax.experimental import pallas as pl
from jax.experimental.pallas import tpu as pltpu
from jax.experimental.pallas import tpu_sc as plsc

B = 4096
F = 26
V = 100000
D = 64

NC = 2
NS = 16
NW = NC * NS
LANES = 16

H0 = 50048
H1 = 49920
VT = 99968
NT = 2 * F


def _gather_half(idx_v, row, ob, lo):
  def body(i, pos):
    v16 = idx_v[pl.ds(i * LANES, LANES)]
    off = v16 - lo
    if lo == 0:
      m = v16 < H0
    else:
      m = v16 >= lo
    g = plsc.load_gather(row, [off], mask=m)
    plsc.store_scatter(ob, [pos], g, mask=m)
    return pos + LANES

  lax.fori_loop(0, B // LANES, body, lax.iota(jnp.int32, LANES), unroll=8)


def _emb2_kernel(xt_hbm, tab_hbm, tail_hbm, out_hbm, idx_v, rowA, rowB, ob,
                 semA, semB):
  w = lax.axis_index("s") * NC + lax.axis_index("c")

  def fire_h0(f, d):
    pltpu.async_copy(tab_hbm.at[f, d, pl.ds(0, H0)], rowA, semA)

  def fire_h1(f, d):
    pltpu.async_copy(tab_hbm.at[f, d, pl.ds(H0, H1)], rowB.at[pl.ds(0, H1)],
                     semB)
    pltpu.async_copy(tail_hbm.at[f, d], rowB.at[pl.ds(H1, 128)], semB)

  fire_h0(0, w)
  fire_h1(0, w)

  def task_body(t, _):
    f = t // 2
    d = (t % 2) * NW + w
    fn = (t + 1) // 2
    dn = ((t + 1) % 2) * NW + w

    @pl.when(t % 2 == 0)
    def _():
      pltpu.sync_copy(xt_hbm.at[f], idx_v)

    pltpu.make_async_copy(tab_hbm.at[f, d, pl.ds(0, H0)], rowA, semA).wait()
    _gather_half(idx_v, rowA, ob, 0)

    @pl.when(t + 1 < NT)
    def _():
      fire_h0(fn, dn)

    pltpu.make_async_copy(tab_hbm.at[f, d, pl.ds(H0, H1)],
                          rowB.at[pl.ds(0, H1)], semB).wait()
    pltpu.make_async_copy(tail_hbm.at[f, d], rowB.at[pl.ds(H1, 128)],
                          semB).wait()
    _gather_half(idx_v, rowB, ob, H0)

    @pl.when(t + 1 < NT)
    def _():
      fire_h1(fn, dn)

    pltpu.sync_copy(ob, out_hbm.at[f, d])
    return 0

  lax.fori_loop(0, NT, task_body, 0)


@jax.jit
def _emb2(xt, tab_p, tab_tail):
  mesh = plsc.VectorSubcoreMesh(
      core_axis_name="c", subcore_axis_name="s", num_cores=NC, num_subcores=NS
  )
  return pl.kernel(
      _emb2_kernel,
      out_type=jax.ShapeDtypeStruct((F, D, B), jnp.float32),
      mesh=mesh,
      scratch_types=[
          pltpu.VMEM((B,), jnp.int32),
          pltpu.VMEM((H0,), jnp.float32),
          pltpu.VMEM((H1 + 128,), jnp.float32),
          pltpu.VMEM((B,), jnp.float32),
          pltpu.SemaphoreType.DMA,
          pltpu.SemaphoreType.DMA,
      ],
      compiler_params=pltpu.CompilerParams(needs_layout_passes=False),
  )(xt, tab_p, tab_tail)


def kernel(X, tables):
  xt = X.T
  tab_p = tables.transpose(0, 2, 1)
  tab_tail = jnp.pad(tables[:, VT:, :].transpose(0, 2, 1), ((0, 0), (0, 0), (0, 96)))
  out_p = _emb2(xt, tab_p, tab_tail)
  return out_p.transpose(2, 0, 1)
